# baseline (speedup 1.0000x reference)
import functools

import jax
import jax.numpy as jnp
import numpy as np
from jax import lax
from jax.experimental import pallas as pl
from jax.experimental.pallas import tpu as pltpu
from jax.experimental.pallas import tpu_sc as plsc

N = 10000
D = 128
H = 128
C = 40
E = 320000

NC = 2
NS = 16
NW = NC * NS
CHUNK = 128
NBUF = 2
NPAD = 10240
ROWS_PER_TILE = NPAD // NS
CT = 80
E_PAD = NW * CHUNK * CT

_BN_INV = float(1.0 / np.sqrt(1.0 + 1e-5))


def _sc_agg_body(with_cnt, *refs):
    if with_cnt:
        h_hbm, src_hbm, dst_hbm, zrows_hbm, zn_hbm, ones_hbm = refs[:6]
        agg_out, cnt_out = refs[6:8]
        scr = refs[8:]
    else:
        h_hbm, src_hbm, dst_hbm, zrows_hbm = refs[:4]
        agg_out = refs[4]
        scr = refs[5:]
    src_idx = scr[0]
    dv = scr[1:1 + NBUF]
    rows = scr[1 + NBUF:1 + 2 * NBUF]
    o = 1 + 2 * NBUF
    if with_cnt:
        ones_v = scr[o]
        o += 1
    gsem = scr[o:o + NBUF]
    dsem = scr[o + NBUF:o + 2 * NBUF]
    o += 2 * NBUF
    acc_sh = scr[o]
    if with_cnt:
        cnt_sh = scr[o + 1]

    c = lax.axis_index("c")
    s = lax.axis_index("s")
    wid = c * NS + s
    r0 = s * ROWS_PER_TILE

    pltpu.sync_copy(src_hbm.at[wid], src_idx)

    def sidx(k):
        return src_idx.at[pl.ds(k * CHUNK, CHUNK)]

    def issue(k, b):
        pltpu.async_copy(h_hbm.at[sidx(k)], rows[b], gsem[b])
        pltpu.async_copy(dst_hbm.at[wid, k], dv[b], dsem[b])

    for p in range(NBUF - 1):
        issue(p, p)

    pltpu.sync_copy(zrows_hbm, acc_sh.at[pl.ds(r0, ROWS_PER_TILE)])
    if with_cnt:
        pltpu.sync_copy(zn_hbm.at[pl.ds(r0, ROWS_PER_TILE)],
                        cnt_sh.at[pl.ds(r0, ROWS_PER_TILE)])
        pltpu.sync_copy(ones_hbm, ones_v)
    plsc.subcore_barrier()

    def body(j, carry):
        for b in range(NBUF):
            k = NBUF * j + b
            pre = k + NBUF - 1

            @pl.when(pre < CT)
            def _():
                issue(pre, (b + NBUF - 1) % NBUF)

            pltpu.make_async_copy(h_hbm.at[sidx(k)], rows[b], gsem[b]).wait()
            pltpu.make_async_copy(dst_hbm.at[wid, k], dv[b], dsem[b]).wait()
            pltpu.sync_copy(rows[b], acc_sh.at[dv[b]], add=True)
            if with_cnt:
                pltpu.sync_copy(ones_v, cnt_sh.at[dv[b]], add=True)
        return carry

    lax.fori_loop(0, CT // NBUF, body, 0)
    plsc.subcore_barrier()

    pltpu.sync_copy(acc_sh.at[pl.ds(r0, ROWS_PER_TILE)],
                    agg_out.at[c, pl.ds(r0, ROWS_PER_TILE)])
    if with_cnt:
        pltpu.sync_copy(cnt_sh.at[pl.ds(r0, ROWS_PER_TILE)],
                        cnt_out.at[c, pl.ds(r0, ROWS_PER_TILE)])


def _make_sc_agg(with_cnt):
    mesh = plsc.VectorSubcoreMesh(core_axis_name="c", subcore_axis_name="s")
    out_type = [jax.ShapeDtypeStruct((NC, NPAD, D), jnp.float32)]
    if with_cnt:
        out_type = out_type + [jax.ShapeDtypeStruct((NC, NPAD), jnp.float32)]
    scratch = [pltpu.VMEM((CT * CHUNK,), jnp.int32)]
    scratch += [pltpu.VMEM((CHUNK,), jnp.int32) for _ in range(NBUF)]
    scratch += [pltpu.VMEM((CHUNK, D), jnp.float32) for _ in range(NBUF)]
    if with_cnt:
        scratch += [pltpu.VMEM((CHUNK,), jnp.float32)]
    scratch += [pltpu.SemaphoreType.DMA for _ in range(2 * NBUF)]
    scratch += [pltpu.VMEM_SHARED((NPAD, D), jnp.float32)]
    if with_cnt:
        scratch += [pltpu.VMEM_SHARED((NPAD,), jnp.float32)]
    return pl.kernel(
        functools.partial(_sc_agg_body, with_cnt),
        out_type=tuple(out_type),
        mesh=mesh,
        scratch_types=tuple(scratch),
    )


_sc_agg_cnt = _make_sc_agg(True)
_sc_agg = _make_sc_agg(False)

BR = 1000


def _tc_layer_body(p_ref, cnt_ref, h_ref, wlT_ref, bl_ref, wrT_ref, g_ref, b_ref, o_ref):
    inv = 1.0 / jnp.maximum(cnt_ref[0] + cnt_ref[1], 1.0)
    mean = (p_ref[0] + p_ref[1]) * inv
    out = (jnp.dot(mean, wlT_ref[...], preferred_element_type=jnp.float32)
           + jnp.dot(h_ref[...], wrT_ref[...], preferred_element_type=jnp.float32)
           + bl_ref[...])
    nrm = jnp.sqrt(jnp.sum(out * out, axis=1, keepdims=True))
    y = out / jnp.maximum(nrm, 1e-12)
    y = y * _BN_INV * g_ref[...] + b_ref[...]
    o_ref[...] = jnp.maximum(y, 0.0)


def _tc_final_body(p_ref, cnt_ref, h_ref, wlT_ref, bl_ref, wrT_ref, g_ref, b_ref,
                   wc1T_ref, bc1_ref, wc2T_ref, bc2_ref, o_ref):
    inv = 1.0 / jnp.maximum(cnt_ref[0] + cnt_ref[1], 1.0)
    mean = (p_ref[0] + p_ref[1]) * inv
    out = (jnp.dot(mean, wlT_ref[...], preferred_element_type=jnp.float32)
           + jnp.dot(h_ref[...], wrT_ref[...], preferred_element_type=jnp.float32)
           + bl_ref[...])
    nrm = jnp.sqrt(jnp.sum(out * out, axis=1, keepdims=True))
    y = out / jnp.maximum(nrm, 1e-12)
    y = y * _BN_INV * g_ref[...] + b_ref[...]
    h3 = jnp.maximum(y, 0.0)
    z = jnp.dot(h3, wc1T_ref[...], preferred_element_type=jnp.float32) + bc1_ref[...]
    z = jnp.maximum(z, 0.0)
    o_ref[...] = jnp.dot(z, wc2T_ref[...], preferred_element_type=jnp.float32) + bc2_ref[...]


def _common_specs():
    return [
        pl.BlockSpec((NC, BR, D), lambda i: (0, i, 0)),
        pl.BlockSpec((NC, BR, 1), lambda i: (0, i, 0)),
        pl.BlockSpec((BR, D), lambda i: (i, 0)),
        pl.BlockSpec((D, H), lambda i: (0, 0)),
        pl.BlockSpec((1, H), lambda i: (0, 0)),
        pl.BlockSpec((D, H), lambda i: (0, 0)),
        pl.BlockSpec((1, H), lambda i: (0, 0)),
        pl.BlockSpec((1, H), lambda i: (0, 0)),
    ]


_tc_layer = pl.pallas_call(
    _tc_layer_body,
    grid=(N // BR,),
    in_specs=_common_specs(),
    out_specs=pl.BlockSpec((BR, H), lambda i: (i, 0)),
    out_shape=jax.ShapeDtypeStruct((N, H), jnp.float32),
)

_tc_final = pl.pallas_call(
    _tc_final_body,
    grid=(N // BR,),
    in_specs=_common_specs() + [
        pl.BlockSpec((H, H // 2), lambda i: (0, 0)),
        pl.BlockSpec((1, H // 2), lambda i: (0, 0)),
        pl.BlockSpec((H // 2, C), lambda i: (0, 0)),
        pl.BlockSpec((1, C), lambda i: (0, 0)),
    ],
    out_specs=pl.BlockSpec((BR, C), lambda i: (i, 0)),
    out_shape=jax.ShapeDtypeStruct((N, C), jnp.float32),
)


def kernel(x, edge_index, Wl0, bl0, Wr0, gamma0, beta0, Wl1, bl1, Wr1, gamma1, beta1,
           Wl2, bl2, Wr2, gamma2, beta2, Wc1, bc1, Wc2, bc2):
    src = edge_index[0]
    dst = edge_index[1]
    pad = E_PAD - E
    src_p = jnp.concatenate(
        [src, (jnp.arange(pad, dtype=jnp.int32) * 131) % N])
    dst_p = jnp.concatenate(
        [dst, N + (jnp.arange(pad, dtype=jnp.int32) % (NPAD - N))])
    src_p = src_p.reshape(NW, CT * CHUNK)
    dst_p = dst_p.reshape(NW, CT, CHUNK)
    zrows = jnp.zeros((ROWS_PER_TILE, D), jnp.float32)
    zn = jnp.zeros((NPAD,), jnp.float32)
    ones_c = jnp.ones((CHUNK,), jnp.float32)

    p0, cnt = _sc_agg_cnt(x, src_p, dst_p, zrows, zn, ones_c)
    cnt = cnt.reshape(NC, NPAD, 1)

    layers = [(Wl0, bl0, Wr0, gamma0, beta0),
              (Wl1, bl1, Wr1, gamma1, beta1),
              (Wl2, bl2, Wr2, gamma2, beta2)]

    h = x
    parts = p0
    for i, (Wl, bl, Wr, g, b) in enumerate(layers):
        common = (parts, cnt, h, Wl.T, bl.reshape(1, H), Wr.T,
                  g.reshape(1, H), b.reshape(1, H))
        if i < 2:
            h = _tc_layer(*common)
            (parts,) = _sc_agg(h, src_p, dst_p, zrows)
        else:
            out = _tc_final(*common, Wc1.T, bc1.reshape(1, H // 2),
                            Wc2.T, bc2.reshape(1, C))
    return out


# speedup vs baseline: 1.0557x; 1.0557x over previous
import functools

import jax
import jax.numpy as jnp
import numpy as np
from jax import lax
from jax.experimental import pallas as pl
from jax.experimental.pallas import tpu as pltpu
from jax.experimental.pallas import tpu_sc as plsc

N = 10000
D = 128
H = 128
C = 40
E = 320000

NC = 2
NS = 16
NW = NC * NS
CHUNK = 32
NBUF = 8
NPAD = 10240
ROWS_PER_TILE = NPAD // NS
CT = 320
E_PAD = NW * CHUNK * CT

_BN_INV = float(1.0 / np.sqrt(1.0 + 1e-5))


def _sc_agg_body(with_cnt, *refs):
    if with_cnt:
        h_hbm, src_hbm, dst_hbm, zrows_hbm, zn_hbm, ones_hbm = refs[:6]
        agg_out, cnt_out = refs[6:8]
        scr = refs[8:]
    else:
        h_hbm, src_hbm, dst_hbm, zrows_hbm = refs[:4]
        agg_out = refs[4]
        scr = refs[5:]
    src_idx = scr[0]
    dv = scr[1:1 + NBUF]
    rows = scr[1 + NBUF:1 + 2 * NBUF]
    o = 1 + 2 * NBUF
    if with_cnt:
        ones_v = scr[o]
        o += 1
    gsem = scr[o:o + NBUF]
    dsem = scr[o + NBUF:o + 2 * NBUF]
    o += 2 * NBUF
    acc_sh = scr[o]
    if with_cnt:
        cnt_sh = scr[o + 1]

    c = lax.axis_index("c")
    s = lax.axis_index("s")
    wid = c * NS + s
    r0 = s * ROWS_PER_TILE

    pltpu.sync_copy(src_hbm.at[wid], src_idx)

    def sidx(k):
        return src_idx.at[pl.ds(k * CHUNK, CHUNK)]

    def issue(k, b):
        pltpu.async_copy(h_hbm.at[sidx(k)], rows[b], gsem[b])
        pltpu.async_copy(dst_hbm.at[wid, k], dv[b], dsem[b])

    for p in range(NBUF - 1):
        issue(p, p)

    pltpu.sync_copy(zrows_hbm, acc_sh.at[pl.ds(r0, ROWS_PER_TILE)])
    if with_cnt:
        pltpu.sync_copy(zn_hbm.at[pl.ds(r0, ROWS_PER_TILE)],
                        cnt_sh.at[pl.ds(r0, ROWS_PER_TILE)])
        pltpu.sync_copy(ones_hbm, ones_v)
    plsc.subcore_barrier()

    def body(j, carry):
        for b in range(NBUF):
            k = NBUF * j + b
            pre = k + NBUF - 1

            @pl.when(pre < CT)
            def _():
                issue(pre, (b + NBUF - 1) % NBUF)

            pltpu.make_async_copy(h_hbm.at[sidx(k)], rows[b], gsem[b]).wait()
            pltpu.make_async_copy(dst_hbm.at[wid, k], dv[b], dsem[b]).wait()
            pltpu.sync_copy(rows[b], acc_sh.at[dv[b]], add=True)
            if with_cnt:
                pltpu.sync_copy(ones_v, cnt_sh.at[dv[b]], add=True)
        return carry

    lax.fori_loop(0, CT // NBUF, body, 0)
    plsc.subcore_barrier()

    pltpu.sync_copy(acc_sh.at[pl.ds(r0, ROWS_PER_TILE)],
                    agg_out.at[c, pl.ds(r0, ROWS_PER_TILE)])
    if with_cnt:
        pltpu.sync_copy(cnt_sh.at[pl.ds(r0, ROWS_PER_TILE)],
                        cnt_out.at[c, pl.ds(r0, ROWS_PER_TILE)])


def _make_sc_agg(with_cnt):
    mesh = plsc.VectorSubcoreMesh(core_axis_name="c", subcore_axis_name="s")
    out_type = [jax.ShapeDtypeStruct((NC, NPAD, D), jnp.float32)]
    if with_cnt:
        out_type = out_type + [jax.ShapeDtypeStruct((NC, NPAD), jnp.float32)]
    scratch = [pltpu.VMEM((CT * CHUNK,), jnp.int32)]
    scratch += [pltpu.VMEM((CHUNK,), jnp.int32) for _ in range(NBUF)]
    scratch += [pltpu.VMEM((CHUNK, D), jnp.float32) for _ in range(NBUF)]
    if with_cnt:
        scratch += [pltpu.VMEM((CHUNK,), jnp.float32)]
    scratch += [pltpu.SemaphoreType.DMA for _ in range(2 * NBUF)]
    scratch += [pltpu.VMEM_SHARED((NPAD, D), jnp.float32)]
    if with_cnt:
        scratch += [pltpu.VMEM_SHARED((NPAD,), jnp.float32)]
    return pl.kernel(
        functools.partial(_sc_agg_body, with_cnt),
        out_type=tuple(out_type),
        mesh=mesh,
        scratch_types=tuple(scratch),
    )


_sc_agg_cnt = _make_sc_agg(True)
_sc_agg = _make_sc_agg(False)

BR = 1000


def _tc_layer_body(p_ref, cnt_ref, h_ref, wlT_ref, bl_ref, wrT_ref, g_ref, b_ref, o_ref):
    inv = 1.0 / jnp.maximum(cnt_ref[0] + cnt_ref[1], 1.0)
    mean = (p_ref[0] + p_ref[1]) * inv
    out = (jnp.dot(mean, wlT_ref[...], preferred_element_type=jnp.float32)
           + jnp.dot(h_ref[...], wrT_ref[...], preferred_element_type=jnp.float32)
           + bl_ref[...])
    nrm = jnp.sqrt(jnp.sum(out * out, axis=1, keepdims=True))
    y = out / jnp.maximum(nrm, 1e-12)
    y = y * _BN_INV * g_ref[...] + b_ref[...]
    o_ref[...] = jnp.maximum(y, 0.0)


def _tc_final_body(p_ref, cnt_ref, h_ref, wlT_ref, bl_ref, wrT_ref, g_ref, b_ref,
                   wc1T_ref, bc1_ref, wc2T_ref, bc2_ref, o_ref):
    inv = 1.0 / jnp.maximum(cnt_ref[0] + cnt_ref[1], 1.0)
    mean = (p_ref[0] + p_ref[1]) * inv
    out = (jnp.dot(mean, wlT_ref[...], preferred_element_type=jnp.float32)
           + jnp.dot(h_ref[...], wrT_ref[...], preferred_element_type=jnp.float32)
           + bl_ref[...])
    nrm = jnp.sqrt(jnp.sum(out * out, axis=1, keepdims=True))
    y = out / jnp.maximum(nrm, 1e-12)
    y = y * _BN_INV * g_ref[...] + b_ref[...]
    h3 = jnp.maximum(y, 0.0)
    z = jnp.dot(h3, wc1T_ref[...], preferred_element_type=jnp.float32) + bc1_ref[...]
    z = jnp.maximum(z, 0.0)
    o_ref[...] = jnp.dot(z, wc2T_ref[...], preferred_element_type=jnp.float32) + bc2_ref[...]


def _common_specs():
    return [
        pl.BlockSpec((NC, BR, D), lambda i: (0, i, 0)),
        pl.BlockSpec((NC, BR, 1), lambda i: (0, i, 0)),
        pl.BlockSpec((BR, D), lambda i: (i, 0)),
        pl.BlockSpec((D, H), lambda i: (0, 0)),
        pl.BlockSpec((1, H), lambda i: (0, 0)),
        pl.BlockSpec((D, H), lambda i: (0, 0)),
        pl.BlockSpec((1, H), lambda i: (0, 0)),
        pl.BlockSpec((1, H), lambda i: (0, 0)),
    ]


_tc_layer = pl.pallas_call(
    _tc_layer_body,
    grid=(N // BR,),
    in_specs=_common_specs(),
    out_specs=pl.BlockSpec((BR, H), lambda i: (i, 0)),
    out_shape=jax.ShapeDtypeStruct((N, H), jnp.float32),
)

_tc_final = pl.pallas_call(
    _tc_final_body,
    grid=(N // BR,),
    in_specs=_common_specs() + [
        pl.BlockSpec((H, H // 2), lambda i: (0, 0)),
        pl.BlockSpec((1, H // 2), lambda i: (0, 0)),
        pl.BlockSpec((H // 2, C), lambda i: (0, 0)),
        pl.BlockSpec((1, C), lambda i: (0, 0)),
    ],
    out_specs=pl.BlockSpec((BR, C), lambda i: (i, 0)),
    out_shape=jax.ShapeDtypeStruct((N, C), jnp.float32),
)


def kernel(x, edge_index, Wl0, bl0, Wr0, gamma0, beta0, Wl1, bl1, Wr1, gamma1, beta1,
           Wl2, bl2, Wr2, gamma2, beta2, Wc1, bc1, Wc2, bc2):
    src = edge_index[0]
    dst = edge_index[1]
    pad = E_PAD - E
    src_p = jnp.concatenate(
        [src, (jnp.arange(pad, dtype=jnp.int32) * 131) % N])
    dst_p = jnp.concatenate(
        [dst, N + (jnp.arange(pad, dtype=jnp.int32) % (NPAD - N))])
    src_p = src_p.reshape(NW, CT * CHUNK)
    dst_p = dst_p.reshape(NW, CT, CHUNK)
    zrows = jnp.zeros((ROWS_PER_TILE, D), jnp.float32)
    zn = jnp.zeros((NPAD,), jnp.float32)
    ones_c = jnp.ones((CHUNK,), jnp.float32)

    p0, cnt = _sc_agg_cnt(x, src_p, dst_p, zrows, zn, ones_c)
    cnt = cnt.reshape(NC, NPAD, 1)

    layers = [(Wl0, bl0, Wr0, gamma0, beta0),
              (Wl1, bl1, Wr1, gamma1, beta1),
              (Wl2, bl2, Wr2, gamma2, beta2)]

    h = x
    parts = p0
    for i, (Wl, bl, Wr, g, b) in enumerate(layers):
        common = (parts, cnt, h, Wl.T, bl.reshape(1, H), Wr.T,
                  g.reshape(1, H), b.reshape(1, H))
        if i < 2:
            h = _tc_layer(*common)
            (parts,) = _sc_agg(h, src_p, dst_p, zrows)
        else:
            out = _tc_final(*common, Wc1.T, bc1.reshape(1, H // 2),
                            Wc2.T, bc2.reshape(1, C))
    return out


# speedup vs baseline: 1.1110x; 1.0524x over previous
import functools

import jax
import jax.numpy as jnp
import numpy as np
from jax import lax
from jax.experimental import pallas as pl
from jax.experimental.pallas import tpu as pltpu
from jax.experimental.pallas import tpu_sc as plsc

N = 10000
D = 128
H = 128
C = 40
E = 320000

NC = 2
NS = 16
NW = NC * NS
CHUNK = 64
NBUF = 4
NPAD = 10240
ROWS_PER_TILE = NPAD // NS
CT = 160
E_PAD = NW * CHUNK * CT

_BN_INV = float(1.0 / np.sqrt(1.0 + 1e-5))


def _sc_agg_body(with_cnt, *refs):
    if with_cnt:
        h_hbm, src_hbm, dst_hbm, zrows_hbm, zn_hbm, ones_hbm = refs[:6]
        agg_out, cnt_out = refs[6:8]
        scr = refs[8:]
    else:
        h_hbm, src_hbm, dst_hbm, zrows_hbm = refs[:4]
        agg_out = refs[4]
        scr = refs[5:]
    src_idx = scr[0]
    dv = scr[1:1 + NBUF]
    rows = scr[1 + NBUF:1 + 2 * NBUF]
    o = 1 + 2 * NBUF
    if with_cnt:
        ones_v = scr[o]
        o += 1
    gsem = scr[o:o + NBUF]
    dsem = scr[o + NBUF:o + 2 * NBUF]
    o += 2 * NBUF
    acc_sh = scr[o]
    if with_cnt:
        cnt_sh = scr[o + 1]

    c = lax.axis_index("c")
    s = lax.axis_index("s")
    wid = c * NS + s
    r0 = s * ROWS_PER_TILE

    pltpu.sync_copy(src_hbm.at[wid], src_idx)

    def sidx(k):
        return src_idx.at[pl.ds(k * CHUNK, CHUNK)]

    def issue(k, b):
        pltpu.async_copy(h_hbm.at[sidx(k)], rows[b], gsem[b])
        pltpu.async_copy(dst_hbm.at[wid, k], dv[b], dsem[b])

    for p in range(NBUF - 1):
        issue(p, p)

    pltpu.sync_copy(zrows_hbm, acc_sh.at[pl.ds(r0, ROWS_PER_TILE)])
    if with_cnt:
        pltpu.sync_copy(zn_hbm.at[pl.ds(r0, ROWS_PER_TILE)],
                        cnt_sh.at[pl.ds(r0, ROWS_PER_TILE)])
        pltpu.sync_copy(ones_hbm, ones_v)
    plsc.subcore_barrier()

    def body(j, carry):
        for b in range(NBUF):
            k = NBUF * j + b
            pre = k + NBUF - 1

            @pl.when(pre < CT)
            def _():
                issue(pre, (b + NBUF - 1) % NBUF)

            pltpu.make_async_copy(h_hbm.at[sidx(k)], rows[b], gsem[b]).wait()
            pltpu.make_async_copy(dst_hbm.at[wid, k], dv[b], dsem[b]).wait()
            pltpu.sync_copy(rows[b], acc_sh.at[dv[b]], add=True)
            if with_cnt:
                pltpu.sync_copy(ones_v, cnt_sh.at[dv[b]], add=True)
        return carry

    lax.fori_loop(0, CT // NBUF, body, 0)
    plsc.subcore_barrier()

    pltpu.sync_copy(acc_sh.at[pl.ds(r0, ROWS_PER_TILE)],
                    agg_out.at[c, pl.ds(r0, ROWS_PER_TILE)])
    if with_cnt:
        pltpu.sync_copy(cnt_sh.at[pl.ds(r0, ROWS_PER_TILE)],
                        cnt_out.at[c, pl.ds(r0, ROWS_PER_TILE)])


def _make_sc_agg(with_cnt):
    mesh = plsc.VectorSubcoreMesh(core_axis_name="c", subcore_axis_name="s")
    out_type = [jax.ShapeDtypeStruct((NC, NPAD, D), jnp.float32)]
    if with_cnt:
        out_type = out_type + [jax.ShapeDtypeStruct((NC, NPAD), jnp.float32)]
    scratch = [pltpu.VMEM((CT * CHUNK,), jnp.int32)]
    scratch += [pltpu.VMEM((CHUNK,), jnp.int32) for _ in range(NBUF)]
    scratch += [pltpu.VMEM((CHUNK, D), jnp.float32) for _ in range(NBUF)]
    if with_cnt:
        scratch += [pltpu.VMEM((CHUNK,), jnp.float32)]
    scratch += [pltpu.SemaphoreType.DMA for _ in range(2 * NBUF)]
    scratch += [pltpu.VMEM_SHARED((NPAD, D), jnp.float32)]
    if with_cnt:
        scratch += [pltpu.VMEM_SHARED((NPAD,), jnp.float32)]
    return pl.kernel(
        functools.partial(_sc_agg_body, with_cnt),
        out_type=tuple(out_type),
        mesh=mesh,
        scratch_types=tuple(scratch),
    )


_sc_agg_cnt = _make_sc_agg(True)
_sc_agg = _make_sc_agg(False)

BR = 1000


def _tc_layer_body(p_ref, cnt_ref, h_ref, wlT_ref, bl_ref, wrT_ref, g_ref, b_ref, o_ref):
    inv = 1.0 / jnp.maximum(cnt_ref[0] + cnt_ref[1], 1.0)
    mean = (p_ref[0] + p_ref[1]) * inv
    out = (jnp.dot(mean, wlT_ref[...], preferred_element_type=jnp.float32)
           + jnp.dot(h_ref[...], wrT_ref[...], preferred_element_type=jnp.float32)
           + bl_ref[...])
    nrm = jnp.sqrt(jnp.sum(out * out, axis=1, keepdims=True))
    y = out / jnp.maximum(nrm, 1e-12)
    y = y * _BN_INV * g_ref[...] + b_ref[...]
    o_ref[...] = jnp.maximum(y, 0.0)


def _tc_final_body(p_ref, cnt_ref, h_ref, wlT_ref, bl_ref, wrT_ref, g_ref, b_ref,
                   wc1T_ref, bc1_ref, wc2T_ref, bc2_ref, o_ref):
    inv = 1.0 / jnp.maximum(cnt_ref[0] + cnt_ref[1], 1.0)
    mean = (p_ref[0] + p_ref[1]) * inv
    out = (jnp.dot(mean, wlT_ref[...], preferred_element_type=jnp.float32)
           + jnp.dot(h_ref[...], wrT_ref[...], preferred_element_type=jnp.float32)
           + bl_ref[...])
    nrm = jnp.sqrt(jnp.sum(out * out, axis=1, keepdims=True))
    y = out / jnp.maximum(nrm, 1e-12)
    y = y * _BN_INV * g_ref[...] + b_ref[...]
    h3 = jnp.maximum(y, 0.0)
    z = jnp.dot(h3, wc1T_ref[...], preferred_element_type=jnp.float32) + bc1_ref[...]
    z = jnp.maximum(z, 0.0)
    o_ref[...] = jnp.dot(z, wc2T_ref[...], preferred_element_type=jnp.float32) + bc2_ref[...]


def _common_specs():
    return [
        pl.BlockSpec((NC, BR, D), lambda i: (0, i, 0)),
        pl.BlockSpec((NC, BR, 1), lambda i: (0, i, 0)),
        pl.BlockSpec((BR, D), lambda i: (i, 0)),
        pl.BlockSpec((D, H), lambda i: (0, 0)),
        pl.BlockSpec((1, H), lambda i: (0, 0)),
        pl.BlockSpec((D, H), lambda i: (0, 0)),
        pl.BlockSpec((1, H), lambda i: (0, 0)),
        pl.BlockSpec((1, H), lambda i: (0, 0)),
    ]


_tc_layer = pl.pallas_call(
    _tc_layer_body,
    grid=(N // BR,),
    in_specs=_common_specs(),
    out_specs=pl.BlockSpec((BR, H), lambda i: (i, 0)),
    out_shape=jax.ShapeDtypeStruct((N, H), jnp.float32),
)

_tc_final = pl.pallas_call(
    _tc_final_body,
    grid=(N // BR,),
    in_specs=_common_specs() + [
        pl.BlockSpec((H, H // 2), lambda i: (0, 0)),
        pl.BlockSpec((1, H // 2), lambda i: (0, 0)),
        pl.BlockSpec((H // 2, C), lambda i: (0, 0)),
        pl.BlockSpec((1, C), lambda i: (0, 0)),
    ],
    out_specs=pl.BlockSpec((BR, C), lambda i: (i, 0)),
    out_shape=jax.ShapeDtypeStruct((N, C), jnp.float32),
)


def kernel(x, edge_index, Wl0, bl0, Wr0, gamma0, beta0, Wl1, bl1, Wr1, gamma1, beta1,
           Wl2, bl2, Wr2, gamma2, beta2, Wc1, bc1, Wc2, bc2):
    src = edge_index[0]
    dst = edge_index[1]
    pad = E_PAD - E
    src_p = jnp.concatenate(
        [src, (jnp.arange(pad, dtype=jnp.int32) * 131) % N])
    dst_p = jnp.concatenate(
        [dst, N + (jnp.arange(pad, dtype=jnp.int32) % (NPAD - N))])
    src_p = src_p.reshape(NW, CT * CHUNK)
    dst_p = dst_p.reshape(NW, CT, CHUNK)
    zrows = jnp.zeros((ROWS_PER_TILE, D), jnp.float32)
    zn = jnp.zeros((NPAD,), jnp.float32)
    ones_c = jnp.ones((CHUNK,), jnp.float32)

    p0, cnt = _sc_agg_cnt(x, src_p, dst_p, zrows, zn, ones_c)
    cnt = cnt.reshape(NC, NPAD, 1)

    layers = [(Wl0, bl0, Wr0, gamma0, beta0),
              (Wl1, bl1, Wr1, gamma1, beta1),
              (Wl2, bl2, Wr2, gamma2, beta2)]

    h = x
    parts = p0
    for i, (Wl, bl, Wr, g, b) in enumerate(layers):
        common = (parts, cnt, h, Wl.T, bl.reshape(1, H), Wr.T,
                  g.reshape(1, H), b.reshape(1, H))
        if i < 2:
            h = _tc_layer(*common)
            (parts,) = _sc_agg(h, src_p, dst_p, zrows)
        else:
            out = _tc_final(*common, Wc1.T, bc1.reshape(1, H // 2),
                            Wc2.T, bc2.reshape(1, C))
    return out
